# baseline (speedup 1.0000x reference)
import functools

import jax
import jax.numpy as jnp
from jax.experimental import pallas as pl

_E = 8
_K = 2
_H = 64


def _moe_kernel(x_ref, wg_ref, w1_ref, w2_ref, out_ref, ep_ref, loss_ref,
                *, nblk, ntok):
    i = pl.program_id(0)
    xb = x_ref[...]

    logits = jnp.dot(xb, wg_ref[...], preferred_element_type=jnp.float32)
    gate = jax.nn.softmax(logits, axis=-1)

    lane = jax.lax.broadcasted_iota(jnp.int32, gate.shape, 1)
    big = jnp.int32(_E)
    m1 = jnp.max(gate, axis=1, keepdims=True)
    i1 = jnp.min(jnp.where(gate == m1, lane, big), axis=1, keepdims=True)
    sel1 = lane == i1
    gate2 = jnp.where(sel1, -jnp.inf, gate)
    m2 = jnp.max(gate2, axis=1, keepdims=True)
    i2 = jnp.min(jnp.where(gate2 == m2, lane, big), axis=1, keepdims=True)
    sel2 = lane == i2
    cw = jnp.where(sel1, m1, 0.0) + jnp.where(sel2, m2, 0.0)

    @pl.when(i == 0)
    def _():
        ep_ref[...] = jnp.zeros_like(ep_ref)

    ep_ref[...] += jnp.sum(gate, axis=0, keepdims=True)

    rep = (jax.lax.broadcasted_iota(jnp.int32, (_E, _E * _H), 1) // _H
           == jax.lax.broadcasted_iota(jnp.int32, (_E, _E * _H), 0)
           ).astype(jnp.float32)
    scale = jnp.dot(cw, rep, preferred_element_type=jnp.float32)

    xb_bf = xb.astype(jnp.bfloat16)
    h = jnp.dot(xb_bf, w1_ref[...], preferred_element_type=jnp.float32)
    h = h * jax.nn.sigmoid(h)
    hs = (h * scale).astype(jnp.bfloat16)
    out_ref[...] = jnp.dot(hs, w2_ref[...],
                           preferred_element_type=jnp.float32)

    @pl.when(i == nblk - 1)
    def _():
        ep = ep_ref[...] / ntok
        m = jnp.mean(ep)
        var = jnp.mean((ep - m) ** 2)
        loss_ref[...] = jnp.full_like(loss_ref, var / (m * m + 1e-10))


def kernel(x, Wg, W1, W2):
    B, S, D = x.shape
    N = B * S
    T = 512
    nblk = N // T

    xf = x.reshape(N, D)
    wg_t = Wg.T
    w1_t = W1.reshape(_E * _H, D).T.astype(jnp.bfloat16)
    w2_r = jnp.transpose(W2, (0, 2, 1)).reshape(_E * _H, D).astype(jnp.bfloat16)

    out, _, loss = pl.pallas_call(
        functools.partial(_moe_kernel, nblk=nblk, ntok=N),
        grid=(nblk,),
        in_specs=[
            pl.BlockSpec((T, D), lambda i: (i, 0)),
            pl.BlockSpec((D, _E), lambda i: (0, 0)),
            pl.BlockSpec((D, _E * _H), lambda i: (0, 0)),
            pl.BlockSpec((_E * _H, D), lambda i: (0, 0)),
        ],
        out_specs=[
            pl.BlockSpec((T, D), lambda i: (i, 0)),
            pl.BlockSpec((1, _E), lambda i: (0, 0)),
            pl.BlockSpec((1, 1), lambda i: (0, 0)),
        ],
        out_shape=[
            jax.ShapeDtypeStruct((N, D), jnp.float32),
            jax.ShapeDtypeStruct((1, _E), jnp.float32),
            jax.ShapeDtypeStruct((1, 1), jnp.float32),
        ],
    )(xf, wg_t, w1_t, w2_r)

    return out.reshape(B, S, D), loss[0, 0]


# speedup vs baseline: 1.1315x; 1.1315x over previous
import functools

import jax
import jax.numpy as jnp
from jax.experimental import pallas as pl

_E = 8
_K = 2
_H = 64


def _moe_kernel(x_ref, wg_ref, w1_ref, w2_ref, out_ref, ep_ref, loss_ref,
                *, nblk, ntok):
    i = pl.program_id(0)
    xb = x_ref[...]

    logits = jnp.dot(xb, wg_ref[...], preferred_element_type=jnp.float32)
    gate = jax.nn.softmax(logits, axis=-1)

    lane = jax.lax.broadcasted_iota(jnp.int32, gate.shape, 1)
    big = jnp.int32(_E)
    m1 = jnp.max(gate, axis=1, keepdims=True)
    i1 = jnp.min(jnp.where(gate == m1, lane, big), axis=1, keepdims=True)
    sel1 = lane == i1
    gate2 = jnp.where(sel1, -jnp.inf, gate)
    m2 = jnp.max(gate2, axis=1, keepdims=True)
    i2 = jnp.min(jnp.where(gate2 == m2, lane, big), axis=1, keepdims=True)
    sel2 = lane == i2
    cw = jnp.where(sel1, m1, 0.0) + jnp.where(sel2, m2, 0.0)

    @pl.when(i == 0)
    def _():
        ep_ref[...] = jnp.zeros_like(ep_ref)

    ep_ref[...] += jnp.sum(gate, axis=0, keepdims=True)

    rep = (jax.lax.broadcasted_iota(jnp.int32, (_E, _E * _H), 1) // _H
           == jax.lax.broadcasted_iota(jnp.int32, (_E, _E * _H), 0)
           ).astype(jnp.float32)
    scale = jnp.dot(cw, rep, preferred_element_type=jnp.float32)

    h = jnp.dot(xb, w1_ref[...], preferred_element_type=jnp.float32)
    h = h * jax.nn.sigmoid(h)
    out_ref[...] = jnp.dot(h * scale, w2_ref[...],
                           preferred_element_type=jnp.float32)

    @pl.when(i == nblk - 1)
    def _():
        ep = ep_ref[...] / ntok
        m = jnp.mean(ep)
        var = jnp.mean((ep - m) ** 2)
        loss_ref[...] = jnp.full_like(loss_ref, var / (m * m + 1e-10))


def kernel(x, Wg, W1, W2):
    B, S, D = x.shape
    N = B * S
    T = 1024
    nblk = N // T

    xf = x.reshape(N, D)
    wg_t = Wg.T
    w1_t = W1.reshape(_E * _H, D).T
    w2_r = jnp.transpose(W2, (0, 2, 1)).reshape(_E * _H, D)

    out, _, loss = pl.pallas_call(
        functools.partial(_moe_kernel, nblk=nblk, ntok=N),
        grid=(nblk,),
        in_specs=[
            pl.BlockSpec((T, D), lambda i: (i, 0)),
            pl.BlockSpec((D, _E), lambda i: (0, 0)),
            pl.BlockSpec((D, _E * _H), lambda i: (0, 0)),
            pl.BlockSpec((_E * _H, D), lambda i: (0, 0)),
        ],
        out_specs=[
            pl.BlockSpec((T, D), lambda i: (i, 0)),
            pl.BlockSpec((1, _E), lambda i: (0, 0)),
            pl.BlockSpec((1, 1), lambda i: (0, 0)),
        ],
        out_shape=[
            jax.ShapeDtypeStruct((N, D), jnp.float32),
            jax.ShapeDtypeStruct((1, _E), jnp.float32),
            jax.ShapeDtypeStruct((1, 1), jnp.float32),
        ],
    )(xf, wg_t, w1_t, w2_r)

    return out.reshape(B, S, D), loss[0, 0]
